# baseline (speedup 1.0000x reference)
import functools

import jax
import jax.numpy as jnp
from jax import lax
from jax.experimental import pallas as pl
from jax.experimental.pallas import tpu as pltpu
from jax.experimental.pallas import tpu_sc as plsc

N = 10000
E = 320000
DIM = 128
NEG_SLOPE = 0.2

NUM_CORES = 2
NUM_SUBCORES = 16
NW = NUM_CORES * NUM_SUBCORES
EPW = E // NW
CH = 80
NCH = EPW // CH
SUP = 2000
CPS = SUP // CH
RPT = 632


def _prep_body(z_ref, w_ref, asrc_ref, adst_ref, h_ref, a_ref, b_ref):
    h = jnp.dot(z_ref[...], w_ref[...], preferred_element_type=jnp.float32)
    h_ref[...] = h
    a_ref[...] = jnp.sum(h * asrc_ref[...], axis=1, keepdims=True)
    b_ref[...] = jnp.sum(h * adst_ref[...], axis=1, keepdims=True)


def _combine_body(h_ref, acc_ref, den_ref, bias_ref, out_ref):
    acc = acc_ref[0] + acc_ref[1]
    den = den_ref[0] + den_ref[1]
    out_ref[...] = (h_ref[...] + acc) / (1.0 + den) + bias_ref[...]


def _leaky(x):
    return jnp.where(x >= 0.0, x, x * NEG_SLOPE)


def _sc_body(h_hbm, a_hbm, b_hbm, src_hbm, dst_hbm,
             acc_out, den_out,
             a_v, b_v, srcb, dstb,
             dstca0, dstcb0, dstca1, dstcb1, wba0, wbb0, wba1, wbb1,
             rows0, rows1, zden, acc_sh, den_sh, sg0, sg1, ss0, ss1):
    c_id = lax.axis_index("c")
    s_id = lax.axis_index("s")
    wid = s_id * NUM_CORES + c_id
    base = wid * EPW

    bufs = ((rows0, wba0, wbb0, dstca0, dstcb0, sg0, ss0),
            (rows1, wba1, wbb1, dstca1, dstcb1, sg1, ss1))

    pltpu.sync_copy(a_hbm, a_v)
    pltpu.sync_copy(b_hbm, b_v)

    def super_load(sidx):
        o = pl.multiple_of(base + sidx * SUP, 8)
        pltpu.sync_copy(src_hbm.at[pl.ds(o, SUP)], srcb)
        pltpu.sync_copy(dst_hbm.at[pl.ds(o, SUP)], dstb)

    super_load(0)

    zv = jnp.zeros((16,), jnp.float32)

    def _zrow(i, _):
        rows0[i // 8, pl.ds((i % 8) * 16, 16)] = zv
        return 0

    lax.fori_loop(0, CH * 8, _zrow, 0)

    def _zden(i, _):
        zden[pl.ds(i * 16, 16)] = zv
        return 0

    lax.fori_loop(0, 64, _zden, 0)

    r0 = pl.multiple_of(jnp.minimum(s_id * RPT, N - RPT), 8)
    for t in range(7):
        pltpu.sync_copy(rows0, acc_sh.at[pl.ds(r0 + t * CH, CH)])
    pltpu.sync_copy(rows0.at[pl.ds(0, 72)], acc_sh.at[pl.ds(r0 + 560, 72)])

    @pl.when(s_id < 10)
    def _():
        pltpu.sync_copy(zden.at[pl.ds(0, 1000)],
                        den_sh.at[pl.ds(s_id * 1000, 1000)])

    plsc.subcore_barrier()

    def wcompute(cj, wba, wbb, dstca, dstcb):
        q = (cj % CPS) * CH
        for g in range(CH // 16):
            s16 = srcb[pl.ds(q + g * 16, 16)]
            d16 = dstb[pl.ds(q + g * 16, 16)]
            if g < 3:
                dstca[pl.ds(g * 16, 16)] = d16
            else:
                dstcb[pl.ds((g - 3) * 16, 16)] = d16
            a_s = plsc.load_gather(a_v, [s16])
            a_d = plsc.load_gather(a_v, [d16])
            b_d = plsc.load_gather(b_v, [d16])
            w = jnp.exp(_leaky(a_s + b_d) - _leaky(a_d + b_d))
            if g < 3:
                wba[pl.ds(g * 16, 16)] = w
            else:
                wbb[pl.ds((g - 3) * 16, 16)] = w

    def gather_desc(cj, rows, sg):
        q = pl.multiple_of((cj % CPS) * CH, 8)
        return pltpu.make_async_copy(h_hbm.at[srcb.at[pl.ds(q, CH)]],
                                     rows, sg)

    def scale_half(rows, wb, glo, ghi):
        @plsc.parallel_loop(glo, ghi, 1, unroll=ghi - glo)
        def _(g):
            w16 = wb[pl.ds((g - glo) * 16, 16)]
            for k in range(16):
                wk = w16[k]
                r = g * 16 + k
                for qq in range(DIM // 16):
                    sl = pl.ds(qq * 16, 16)
                    rows[r, sl] = rows[r, sl] * wk

    def scatter_waits(rows, wba, wbb, dstca, dstcb, ss):
        pltpu.make_async_copy(rows.at[pl.ds(0, 48)],
                              acc_sh.at[dstca], ss).wait()
        pltpu.make_async_copy(rows.at[pl.ds(48, 32)],
                              acc_sh.at[dstcb], ss).wait()
        pltpu.make_async_copy(wba, den_sh.at[dstca], ss).wait()
        pltpu.make_async_copy(wbb, den_sh.at[dstcb], ss).wait()

    def iter_body(cj, P, first=False, has_next=True, may_bound=True):
        rowsP, wbaP, wbbP, dstcaP, dstcbP, sgP, ssP = bufs[P]
        rowsQ, wbaQ, wbbQ, dstcaQ, dstcbQ, sgQ, ssQ = bufs[1 - P]
        if not first:
            scatter_waits(rowsQ, wbaQ, wbbQ, dstcaQ, dstcbQ, ssQ)
        if may_bound:
            bound = ((cj + 1) % CPS) == 0

            @pl.when(bound)
            def _():
                gather_desc(cj, rowsP, sgP).wait()
                super_load((cj + 1) // CPS)

            if has_next:
                pltpu.async_copy(h_hbm.at[srcb.at[
                    pl.ds(pl.multiple_of(((cj + 1) % CPS) * CH, 8), CH)]],
                    rowsQ, sgQ)

            @pl.when(jnp.logical_not(bound))
            def _():
                gather_desc(cj, rowsP, sgP).wait()
        else:
            if has_next:
                pltpu.async_copy(h_hbm.at[srcb.at[
                    pl.ds(pl.multiple_of(((cj + 1) % CPS) * CH, 8), CH)]],
                    rowsQ, sgQ)
            gather_desc(cj, rowsP, sgP).wait()
        scale_half(rowsP, wbaP, 0, 3)
        pltpu.async_copy(rowsP.at[pl.ds(0, 48)], acc_sh.at[dstcaP],
                         ssP, add=True)
        scale_half(rowsP, wbbP, 3, 5)
        pltpu.async_copy(rowsP.at[pl.ds(48, 32)], acc_sh.at[dstcbP],
                         ssP, add=True)
        pltpu.async_copy(wbaP, den_sh.at[dstcaP], ssP, add=True)
        pltpu.async_copy(wbbP, den_sh.at[dstcbP], ssP, add=True)
        if has_next:
            wcompute(cj + 1, wbaQ, wbbQ, dstcaQ, dstcbQ)

    wcompute(0, wba0, wbb0, dstca0, dstcb0)
    pltpu.async_copy(h_hbm.at[srcb.at[pl.ds(0, CH)]], rows0, sg0)
    iter_body(0, 0, first=True, may_bound=False)

    def _pair(p, _):
        iter_body(2 * p + 1, 1)
        iter_body(2 * p + 2, 0)
        return 0

    lax.fori_loop(0, (NCH - 3) // 2, _pair, 0)

    iter_body(NCH - 2, 1, may_bound=False)
    iter_body(NCH - 1, 0, has_next=False, may_bound=False)
    scatter_waits(rows0, wba0, wbb0, dstca0, dstcb0, ss0)

    plsc.subcore_barrier()

    for t in range(7):
        pltpu.sync_copy(acc_sh.at[pl.ds(r0 + t * CH, CH)], rows0)
        pltpu.sync_copy(rows0, acc_out.at[c_id, pl.ds(r0 + t * CH, CH)])
    pltpu.sync_copy(acc_sh.at[pl.ds(r0 + 560, 72)], rows0.at[pl.ds(0, 72)])
    pltpu.sync_copy(rows0.at[pl.ds(0, 72)], acc_out.at[c_id, pl.ds(r0 + 560, 72)])

    @pl.when(s_id < 10)
    def _():
        pltpu.sync_copy(den_sh.at[pl.ds(s_id * 1000, 1000)],
                        zden.at[pl.ds(0, 1000)])
        pltpu.sync_copy(zden.at[pl.ds(0, 1000)],
                        den_out.at[pl.ds(c_id * N + s_id * 1000, 1000)])


_sc_edge_pass = functools.partial(
    pl.kernel,
    out_type=(jax.ShapeDtypeStruct((NUM_CORES, N, DIM), jnp.float32),
              jax.ShapeDtypeStruct((NUM_CORES * N,), jnp.float32)),
    mesh=plsc.VectorSubcoreMesh(core_axis_name="c", subcore_axis_name="s"),
    scratch_types=[
        pltpu.VMEM((N,), jnp.float32),
        pltpu.VMEM((N,), jnp.float32),
        pltpu.VMEM((SUP,), jnp.int32),
        pltpu.VMEM((SUP,), jnp.int32),
        pltpu.VMEM((48,), jnp.int32),
        pltpu.VMEM((32,), jnp.int32),
        pltpu.VMEM((48,), jnp.int32),
        pltpu.VMEM((32,), jnp.int32),
        pltpu.VMEM((48,), jnp.float32),
        pltpu.VMEM((32,), jnp.float32),
        pltpu.VMEM((48,), jnp.float32),
        pltpu.VMEM((32,), jnp.float32),
        pltpu.VMEM((CH, DIM), jnp.float32),
        pltpu.VMEM((CH, DIM), jnp.float32),
        pltpu.VMEM((1024,), jnp.float32),
        pltpu.VMEM_SHARED((N, DIM), jnp.float32),
        pltpu.VMEM_SHARED((N,), jnp.float32),
        pltpu.SemaphoreType.DMA,
        pltpu.SemaphoreType.DMA,
        pltpu.SemaphoreType.DMA,
        pltpu.SemaphoreType.DMA,
    ],
    compiler_params=pltpu.CompilerParams(needs_layout_passes=False),
)(_sc_body)


def kernel(z, edge_index, W, att_src, att_dst, bias):
    src = edge_index[0]
    dst = edge_index[1]

    h, a, b = pl.pallas_call(
        _prep_body,
        out_shape=(jax.ShapeDtypeStruct((N, DIM), jnp.float32),
                   jax.ShapeDtypeStruct((N, 1), jnp.float32),
                   jax.ShapeDtypeStruct((N, 1), jnp.float32)),
    )(z, W, att_src.reshape(1, DIM), att_dst.reshape(1, DIM))

    acc, den = _sc_edge_pass(h, a.reshape(N), b.reshape(N), src, dst)

    out = pl.pallas_call(
        _combine_body,
        out_shape=jax.ShapeDtypeStruct((N, DIM), jnp.float32),
    )(h, acc, den.reshape(NUM_CORES, N, 1), bias.reshape(1, DIM))
    return out


# speedup vs baseline: 1.1882x; 1.1882x over previous
import functools

import jax
import jax.numpy as jnp
from jax import lax
from jax.experimental import pallas as pl
from jax.experimental.pallas import tpu as pltpu
from jax.experimental.pallas import tpu_sc as plsc

N = 10000
E = 320000
DIM = 128
NEG_SLOPE = 0.2

NUM_CORES = 2
NUM_SUBCORES = 16
NW = NUM_CORES * NUM_SUBCORES
EPW = E // NW
CH = 80
NCH = EPW // CH
SUP = 2000
CPS = SUP // CH
RPT = 632


def _prep_body(z_ref, w_ref, asrc_ref, adst_ref, h_ref, a_ref, b_ref):
    h = jnp.dot(z_ref[...], w_ref[...], preferred_element_type=jnp.float32)
    h_ref[...] = h
    a_ref[...] = jnp.sum(h * asrc_ref[...], axis=1, keepdims=True)
    b_ref[...] = jnp.sum(h * adst_ref[...], axis=1, keepdims=True)


def _combine_body(h_ref, acc_ref, den_ref, bias_ref, out_ref):
    acc = acc_ref[0] + acc_ref[1]
    den = den_ref[0] + den_ref[1]
    out_ref[...] = (h_ref[...] + acc) / (1.0 + den) + bias_ref[...]


def _leaky(x):
    return jnp.where(x >= 0.0, x, x * NEG_SLOPE)


def _sc_body(h_hbm, a_hbm, b_hbm, src_hbm, dst_hbm,
             acc_out, den_out,
             a_v, b_v, srcb, dstb,
             dstca0, dstcb0, dstca1, dstcb1, wba0, wbb0, wba1, wbb1,
             rows0, rows1, zden, acc_sh, den_sh, sg0, sg1, ss0, ss1):
    c_id = lax.axis_index("c")
    s_id = lax.axis_index("s")
    wid = s_id * NUM_CORES + c_id
    base = wid * EPW

    bufs = ((rows0, wba0, wbb0, dstca0, dstcb0, sg0, ss0),
            (rows1, wba1, wbb1, dstca1, dstcb1, sg1, ss1))

    pltpu.sync_copy(a_hbm, a_v)
    pltpu.sync_copy(b_hbm, b_v)

    def super_load(sidx):
        o = pl.multiple_of(base + sidx * SUP, 8)
        pltpu.sync_copy(src_hbm.at[pl.ds(o, SUP)], srcb)
        pltpu.sync_copy(dst_hbm.at[pl.ds(o, SUP)], dstb)

    super_load(0)

    zv = jnp.zeros((16,), jnp.float32)

    def _zrow(i, _):
        rows0[i // 8, pl.ds((i % 8) * 16, 16)] = zv
        return 0

    lax.fori_loop(0, CH * 8, _zrow, 0)

    def _zden(i, _):
        zden[pl.ds(i * 16, 16)] = zv
        return 0

    lax.fori_loop(0, 64, _zden, 0)

    r0 = pl.multiple_of(jnp.minimum(s_id * RPT, N - RPT), 8)
    for t in range(7):
        pltpu.sync_copy(rows0, acc_sh.at[pl.ds(r0 + t * CH, CH)])
    pltpu.sync_copy(rows0.at[pl.ds(0, 72)], acc_sh.at[pl.ds(r0 + 560, 72)])

    @pl.when(s_id < 10)
    def _():
        pltpu.sync_copy(zden.at[pl.ds(0, 1000)],
                        den_sh.at[pl.ds(s_id * 1000, 1000)])

    plsc.subcore_barrier()

    def wcompute(cj, wba, wbb, dstca, dstcb):
        q = (cj % CPS) * CH
        for g in range(CH // 16):
            s16 = srcb[pl.ds(q + g * 16, 16)]
            d16 = dstb[pl.ds(q + g * 16, 16)]
            if g < 3:
                dstca[pl.ds(g * 16, 16)] = d16
            else:
                dstcb[pl.ds((g - 3) * 16, 16)] = d16
            a_s = plsc.load_gather(a_v, [s16])
            a_d = plsc.load_gather(a_v, [d16])
            b_d = plsc.load_gather(b_v, [d16])
            w = jnp.exp(_leaky(a_s + b_d) - _leaky(a_d + b_d))
            if g < 3:
                wba[pl.ds(g * 16, 16)] = w
            else:
                wbb[pl.ds((g - 3) * 16, 16)] = w

    def gather_desc(cj, rows, sg):
        q = pl.multiple_of((cj % CPS) * CH, 8)
        return pltpu.make_async_copy(h_hbm.at[srcb.at[pl.ds(q, CH)]],
                                     rows, sg)

    def scale_half(rows, wb, glo, ghi):
        @plsc.parallel_loop(glo * 16, ghi * 16, 1, unroll=4)
        def _(k):
            wk = plsc.load_gather(wb, [jnp.full((16,), k - glo * 16,
                                                jnp.int32)])
            for qq in range(DIM // 16):
                sl = pl.ds(qq * 16, 16)
                rows[k, sl] = rows[k, sl] * wk

    def scatter_waits(rows, wba, wbb, dstca, dstcb, ss):
        pltpu.make_async_copy(rows.at[pl.ds(0, 48)],
                              acc_sh.at[dstca], ss).wait()
        pltpu.make_async_copy(rows.at[pl.ds(48, 32)],
                              acc_sh.at[dstcb], ss).wait()
        pltpu.make_async_copy(wba, den_sh.at[dstca], ss).wait()
        pltpu.make_async_copy(wbb, den_sh.at[dstcb], ss).wait()

    def iter_body(cj, P, first=False, has_next=True, may_bound=True):
        rowsP, wbaP, wbbP, dstcaP, dstcbP, sgP, ssP = bufs[P]
        rowsQ, wbaQ, wbbQ, dstcaQ, dstcbQ, sgQ, ssQ = bufs[1 - P]
        if not first:
            scatter_waits(rowsQ, wbaQ, wbbQ, dstcaQ, dstcbQ, ssQ)
        if may_bound:
            bound = ((cj + 1) % CPS) == 0

            @pl.when(bound)
            def _():
                gather_desc(cj, rowsP, sgP).wait()
                super_load((cj + 1) // CPS)

            if has_next:
                pltpu.async_copy(h_hbm.at[srcb.at[
                    pl.ds(pl.multiple_of(((cj + 1) % CPS) * CH, 8), CH)]],
                    rowsQ, sgQ)

            @pl.when(jnp.logical_not(bound))
            def _():
                gather_desc(cj, rowsP, sgP).wait()
        else:
            if has_next:
                pltpu.async_copy(h_hbm.at[srcb.at[
                    pl.ds(pl.multiple_of(((cj + 1) % CPS) * CH, 8), CH)]],
                    rowsQ, sgQ)
            gather_desc(cj, rowsP, sgP).wait()
        scale_half(rowsP, wbaP, 0, 3)
        pltpu.async_copy(rowsP.at[pl.ds(0, 48)], acc_sh.at[dstcaP],
                         ssP, add=True)
        scale_half(rowsP, wbbP, 3, 5)
        pltpu.async_copy(rowsP.at[pl.ds(48, 32)], acc_sh.at[dstcbP],
                         ssP, add=True)
        pltpu.async_copy(wbaP, den_sh.at[dstcaP], ssP, add=True)
        pltpu.async_copy(wbbP, den_sh.at[dstcbP], ssP, add=True)
        if has_next:
            wcompute(cj + 1, wbaQ, wbbQ, dstcaQ, dstcbQ)

    wcompute(0, wba0, wbb0, dstca0, dstcb0)
    pltpu.async_copy(h_hbm.at[srcb.at[pl.ds(0, CH)]], rows0, sg0)
    iter_body(0, 0, first=True, may_bound=False)

    def _pair(p, _):
        iter_body(2 * p + 1, 1)
        iter_body(2 * p + 2, 0)
        return 0

    lax.fori_loop(0, (NCH - 3) // 2, _pair, 0)

    iter_body(NCH - 2, 1, may_bound=False)
    iter_body(NCH - 1, 0, has_next=False, may_bound=False)
    scatter_waits(rows0, wba0, wbb0, dstca0, dstcb0, ss0)

    plsc.subcore_barrier()

    for t in range(7):
        pltpu.sync_copy(acc_sh.at[pl.ds(r0 + t * CH, CH)], rows0)
        pltpu.sync_copy(rows0, acc_out.at[c_id, pl.ds(r0 + t * CH, CH)])
    pltpu.sync_copy(acc_sh.at[pl.ds(r0 + 560, 72)], rows0.at[pl.ds(0, 72)])
    pltpu.sync_copy(rows0.at[pl.ds(0, 72)], acc_out.at[c_id, pl.ds(r0 + 560, 72)])

    @pl.when(s_id < 10)
    def _():
        pltpu.sync_copy(den_sh.at[pl.ds(s_id * 1000, 1000)],
                        zden.at[pl.ds(0, 1000)])
        pltpu.sync_copy(zden.at[pl.ds(0, 1000)],
                        den_out.at[pl.ds(c_id * N + s_id * 1000, 1000)])


_sc_edge_pass = functools.partial(
    pl.kernel,
    out_type=(jax.ShapeDtypeStruct((NUM_CORES, N, DIM), jnp.float32),
              jax.ShapeDtypeStruct((NUM_CORES * N,), jnp.float32)),
    mesh=plsc.VectorSubcoreMesh(core_axis_name="c", subcore_axis_name="s"),
    scratch_types=[
        pltpu.VMEM((N,), jnp.float32),
        pltpu.VMEM((N,), jnp.float32),
        pltpu.VMEM((SUP,), jnp.int32),
        pltpu.VMEM((SUP,), jnp.int32),
        pltpu.VMEM((48,), jnp.int32),
        pltpu.VMEM((32,), jnp.int32),
        pltpu.VMEM((48,), jnp.int32),
        pltpu.VMEM((32,), jnp.int32),
        pltpu.VMEM((48,), jnp.float32),
        pltpu.VMEM((32,), jnp.float32),
        pltpu.VMEM((48,), jnp.float32),
        pltpu.VMEM((32,), jnp.float32),
        pltpu.VMEM((CH, DIM), jnp.float32),
        pltpu.VMEM((CH, DIM), jnp.float32),
        pltpu.VMEM((1024,), jnp.float32),
        pltpu.VMEM_SHARED((N, DIM), jnp.float32),
        pltpu.VMEM_SHARED((N,), jnp.float32),
        pltpu.SemaphoreType.DMA,
        pltpu.SemaphoreType.DMA,
        pltpu.SemaphoreType.DMA,
        pltpu.SemaphoreType.DMA,
    ],
    compiler_params=pltpu.CompilerParams(needs_layout_passes=False),
)(_sc_body)


def kernel(z, edge_index, W, att_src, att_dst, bias):
    src = edge_index[0]
    dst = edge_index[1]

    h, a, b = pl.pallas_call(
        _prep_body,
        out_shape=(jax.ShapeDtypeStruct((N, DIM), jnp.float32),
                   jax.ShapeDtypeStruct((N, 1), jnp.float32),
                   jax.ShapeDtypeStruct((N, 1), jnp.float32)),
    )(z, W, att_src.reshape(1, DIM), att_dst.reshape(1, DIM))

    acc, den = _sc_edge_pass(h, a.reshape(N), b.reshape(N), src, dst)

    out = pl.pallas_call(
        _combine_body,
        out_shape=jax.ShapeDtypeStruct((N, DIM), jnp.float32),
    )(h, acc, den.reshape(NUM_CORES, N, 1), bias.reshape(1, DIM))
    return out


# speedup vs baseline: 1.2537x; 1.0551x over previous
import functools

import jax
import jax.numpy as jnp
from jax import lax
from jax.experimental import pallas as pl
from jax.experimental.pallas import tpu as pltpu
from jax.experimental.pallas import tpu_sc as plsc

N = 10000
E = 320000
DIM = 128
NEG_SLOPE = 0.2

NUM_CORES = 2
NUM_SUBCORES = 16
NW = NUM_CORES * NUM_SUBCORES
EPW = E // NW
CH = 80
NCH = EPW // CH
SUP = 2000
CPS = SUP // CH
RPT = 632


def _prep_body(z_ref, w_ref, asrc_ref, adst_ref, h_ref, a_ref, b_ref):
    h = jnp.dot(z_ref[...], w_ref[...], preferred_element_type=jnp.float32)
    h_ref[...] = h
    a_ref[...] = jnp.sum(h * asrc_ref[...], axis=1, keepdims=True)
    b_ref[...] = jnp.sum(h * adst_ref[...], axis=1, keepdims=True)


def _edge_split_body(ei_ref, src_ref, dst_ref):
    src_ref[...] = ei_ref[0]
    dst_ref[...] = ei_ref[1]


def _combine_body(h_ref, acc_ref, den_ref, bias_ref, out_ref):
    acc = acc_ref[0] + acc_ref[1]
    den = den_ref[0] + den_ref[1]
    out_ref[...] = (h_ref[...] + acc) / (1.0 + den) + bias_ref[...]


def _leaky(x):
    return jnp.where(x >= 0.0, x, x * NEG_SLOPE)


def _sc_body(h_hbm, a_hbm, b_hbm, src_hbm, dst_hbm,
             acc_out, den_out,
             a_v, b_v, srcb, dstb,
             dstca0, dstcb0, dstca1, dstcb1, wba0, wbb0, wba1, wbb1,
             rows0, rows1, zden, acc_sh, den_sh, sg0, sg1, ss0, ss1):
    c_id = lax.axis_index("c")
    s_id = lax.axis_index("s")
    wid = s_id * NUM_CORES + c_id
    base = wid * EPW

    bufs = ((rows0, wba0, wbb0, dstca0, dstcb0, sg0, ss0),
            (rows1, wba1, wbb1, dstca1, dstcb1, sg1, ss1))

    pltpu.sync_copy(a_hbm, a_v)
    pltpu.sync_copy(b_hbm, b_v)

    def super_load(sidx):
        o = pl.multiple_of(base + sidx * SUP, 8)
        pltpu.sync_copy(src_hbm.at[pl.ds(o, SUP)], srcb)
        pltpu.sync_copy(dst_hbm.at[pl.ds(o, SUP)], dstb)

    super_load(0)

    zv = jnp.zeros((16,), jnp.float32)

    def _zrow(i, _):
        rows0[i // 8, pl.ds((i % 8) * 16, 16)] = zv
        return 0

    lax.fori_loop(0, CH * 8, _zrow, 0)

    def _zden(i, _):
        zden[pl.ds(i * 16, 16)] = zv
        return 0

    lax.fori_loop(0, 64, _zden, 0)

    r0 = pl.multiple_of(jnp.minimum(s_id * RPT, N - RPT), 8)
    for t in range(7):
        pltpu.sync_copy(rows0, acc_sh.at[pl.ds(r0 + t * CH, CH)])
    pltpu.sync_copy(rows0.at[pl.ds(0, 72)], acc_sh.at[pl.ds(r0 + 560, 72)])

    @pl.when(s_id < 10)
    def _():
        pltpu.sync_copy(zden.at[pl.ds(0, 1000)],
                        den_sh.at[pl.ds(s_id * 1000, 1000)])

    plsc.subcore_barrier()

    def wcompute(cj, wba, wbb, dstca, dstcb):
        q = (cj % CPS) * CH
        for g in range(CH // 16):
            s16 = srcb[pl.ds(q + g * 16, 16)]
            d16 = dstb[pl.ds(q + g * 16, 16)]
            if g < 3:
                dstca[pl.ds(g * 16, 16)] = d16
            else:
                dstcb[pl.ds((g - 3) * 16, 16)] = d16
            a_s = plsc.load_gather(a_v, [s16])
            a_d = plsc.load_gather(a_v, [d16])
            b_d = plsc.load_gather(b_v, [d16])
            w = jnp.exp(_leaky(a_s + b_d) - _leaky(a_d + b_d))
            if g < 3:
                wba[pl.ds(g * 16, 16)] = w
            else:
                wbb[pl.ds((g - 3) * 16, 16)] = w

    def gather_desc(cj, rows, sg):
        q = pl.multiple_of((cj % CPS) * CH, 8)
        return pltpu.make_async_copy(h_hbm.at[srcb.at[pl.ds(q, CH)]],
                                     rows, sg)

    def scale_half(rows, wb, glo, ghi):
        @plsc.parallel_loop(glo * 16, ghi * 16, 1, unroll=4)
        def _(k):
            wk = plsc.load_gather(wb, [jnp.full((16,), k - glo * 16,
                                                jnp.int32)])
            for qq in range(DIM // 16):
                sl = pl.ds(qq * 16, 16)
                rows[k, sl] = rows[k, sl] * wk

    def scatter_waits(rows, wba, wbb, dstca, dstcb, ss):
        pltpu.make_async_copy(rows.at[pl.ds(0, 48)],
                              acc_sh.at[dstca], ss).wait()
        pltpu.make_async_copy(rows.at[pl.ds(48, 32)],
                              acc_sh.at[dstcb], ss).wait()
        pltpu.make_async_copy(wba, den_sh.at[dstca], ss).wait()
        pltpu.make_async_copy(wbb, den_sh.at[dstcb], ss).wait()

    def iter_body(cj, P, first=False, has_next=True, may_bound=True):
        rowsP, wbaP, wbbP, dstcaP, dstcbP, sgP, ssP = bufs[P]
        rowsQ, wbaQ, wbbQ, dstcaQ, dstcbQ, sgQ, ssQ = bufs[1 - P]
        if not first:
            scatter_waits(rowsQ, wbaQ, wbbQ, dstcaQ, dstcbQ, ssQ)
        if may_bound:
            bound = ((cj + 1) % CPS) == 0

            @pl.when(bound)
            def _():
                gather_desc(cj, rowsP, sgP).wait()
                super_load((cj + 1) // CPS)

            if has_next:
                pltpu.async_copy(h_hbm.at[srcb.at[
                    pl.ds(pl.multiple_of(((cj + 1) % CPS) * CH, 8), CH)]],
                    rowsQ, sgQ)

            @pl.when(jnp.logical_not(bound))
            def _():
                gather_desc(cj, rowsP, sgP).wait()
        else:
            if has_next:
                pltpu.async_copy(h_hbm.at[srcb.at[
                    pl.ds(pl.multiple_of(((cj + 1) % CPS) * CH, 8), CH)]],
                    rowsQ, sgQ)
            gather_desc(cj, rowsP, sgP).wait()
        scale_half(rowsP, wbaP, 0, 3)
        pltpu.async_copy(rowsP.at[pl.ds(0, 48)], acc_sh.at[dstcaP],
                         ssP, add=True)
        scale_half(rowsP, wbbP, 3, 5)
        pltpu.async_copy(rowsP.at[pl.ds(48, 32)], acc_sh.at[dstcbP],
                         ssP, add=True)
        pltpu.async_copy(wbaP, den_sh.at[dstcaP], ssP, add=True)
        pltpu.async_copy(wbbP, den_sh.at[dstcbP], ssP, add=True)
        if has_next:
            wcompute(cj + 1, wbaQ, wbbQ, dstcaQ, dstcbQ)

    wcompute(0, wba0, wbb0, dstca0, dstcb0)
    pltpu.async_copy(h_hbm.at[srcb.at[pl.ds(0, CH)]], rows0, sg0)
    iter_body(0, 0, first=True, may_bound=False)

    def _pair(p, _):
        iter_body(2 * p + 1, 1)
        iter_body(2 * p + 2, 0)
        return 0

    lax.fori_loop(0, (NCH - 3) // 2, _pair, 0)

    iter_body(NCH - 2, 1, may_bound=False)
    iter_body(NCH - 1, 0, has_next=False, may_bound=False)
    scatter_waits(rows0, wba0, wbb0, dstca0, dstcb0, ss0)

    plsc.subcore_barrier()

    for t in range(7):
        pltpu.sync_copy(acc_sh.at[pl.ds(r0 + t * CH, CH)], rows0)
        pltpu.sync_copy(rows0, acc_out.at[c_id, pl.ds(r0 + t * CH, CH)])
    pltpu.sync_copy(acc_sh.at[pl.ds(r0 + 560, 72)], rows0.at[pl.ds(0, 72)])
    pltpu.sync_copy(rows0.at[pl.ds(0, 72)], acc_out.at[c_id, pl.ds(r0 + 560, 72)])

    @pl.when(s_id < 10)
    def _():
        pltpu.sync_copy(den_sh.at[pl.ds(s_id * 1000, 1000)],
                        zden.at[pl.ds(0, 1000)])
        pltpu.sync_copy(zden.at[pl.ds(0, 1000)],
                        den_out.at[pl.ds(c_id * N + s_id * 1000, 1000)])


_sc_edge_pass = functools.partial(
    pl.kernel,
    out_type=(jax.ShapeDtypeStruct((NUM_CORES, N, DIM), jnp.float32),
              jax.ShapeDtypeStruct((NUM_CORES * N,), jnp.float32)),
    mesh=plsc.VectorSubcoreMesh(core_axis_name="c", subcore_axis_name="s"),
    scratch_types=[
        pltpu.VMEM((N,), jnp.float32),
        pltpu.VMEM((N,), jnp.float32),
        pltpu.VMEM((SUP,), jnp.int32),
        pltpu.VMEM((SUP,), jnp.int32),
        pltpu.VMEM((48,), jnp.int32),
        pltpu.VMEM((32,), jnp.int32),
        pltpu.VMEM((48,), jnp.int32),
        pltpu.VMEM((32,), jnp.int32),
        pltpu.VMEM((48,), jnp.float32),
        pltpu.VMEM((32,), jnp.float32),
        pltpu.VMEM((48,), jnp.float32),
        pltpu.VMEM((32,), jnp.float32),
        pltpu.VMEM((CH, DIM), jnp.float32),
        pltpu.VMEM((CH, DIM), jnp.float32),
        pltpu.VMEM((1024,), jnp.float32),
        pltpu.VMEM_SHARED((N, DIM), jnp.float32),
        pltpu.VMEM_SHARED((N,), jnp.float32),
        pltpu.SemaphoreType.DMA,
        pltpu.SemaphoreType.DMA,
        pltpu.SemaphoreType.DMA,
        pltpu.SemaphoreType.DMA,
    ],
    compiler_params=pltpu.CompilerParams(needs_layout_passes=False),
)(_sc_body)


def kernel(z, edge_index, W, att_src, att_dst, bias):
    h, a, b = pl.pallas_call(
        _prep_body,
        out_shape=(jax.ShapeDtypeStruct((N, DIM), jnp.float32),
                   jax.ShapeDtypeStruct((N, 1), jnp.float32),
                   jax.ShapeDtypeStruct((N, 1), jnp.float32)),
    )(z, W, att_src.reshape(1, DIM), att_dst.reshape(1, DIM))

    src, dst = pl.pallas_call(
        _edge_split_body,
        out_shape=(jax.ShapeDtypeStruct((E,), jnp.int32),
                   jax.ShapeDtypeStruct((E,), jnp.int32)),
    )(edge_index)

    acc, den = _sc_edge_pass(h, a.reshape(N), b.reshape(N), src, dst)

    out = pl.pallas_call(
        _combine_body,
        out_shape=jax.ShapeDtypeStruct((N, DIM), jnp.float32),
    )(h, acc, den.reshape(NUM_CORES, N, 1), bias.reshape(1, DIM))
    return out


# speedup vs baseline: 1.2629x; 1.0073x over previous
import functools

import jax
import jax.numpy as jnp
from jax import lax
from jax.experimental import pallas as pl
from jax.experimental.pallas import tpu as pltpu
from jax.experimental.pallas import tpu_sc as plsc

N = 10000
E = 320000
DIM = 128
NEG_SLOPE = 0.2

NUM_CORES = 2
NUM_SUBCORES = 16
NW = NUM_CORES * NUM_SUBCORES
EPW = E // NW
CH = 80
NCH = EPW // CH
SUP = 2000
CPS = SUP // CH
RPT = 632


def _prep_body(z_ref, w_ref, asrc_ref, adst_ref, ei_ref,
               h_ref, a_ref, b_ref, src_ref, dst_ref):
    h = jnp.dot(z_ref[...], w_ref[...], preferred_element_type=jnp.float32)
    h_ref[...] = h
    a_ref[...] = jnp.sum(h * asrc_ref[...], axis=1, keepdims=True)
    b_ref[...] = jnp.sum(h * adst_ref[...], axis=1, keepdims=True)
    src_ref[...] = ei_ref[0]
    dst_ref[...] = ei_ref[1]


def _combine_body(h_ref, acc_ref, den_ref, bias_ref, out_ref):
    acc = acc_ref[0] + acc_ref[1]
    den = den_ref[0] + den_ref[1]
    out_ref[...] = (h_ref[...] + acc) / (1.0 + den) + bias_ref[...]


def _leaky(x):
    return jnp.where(x >= 0.0, x, x * NEG_SLOPE)


def _sc_body(h_hbm, a_hbm, b_hbm, src_hbm, dst_hbm,
             acc_out, den_out,
             a_v, b_v, srcb, dstb,
             dstca0, dstcb0, dstca1, dstcb1, wba0, wbb0, wba1, wbb1,
             rows0, rows1, zden, acc_sh, den_sh, sg0, sg1, ss0, ss1):
    c_id = lax.axis_index("c")
    s_id = lax.axis_index("s")
    wid = s_id * NUM_CORES + c_id
    base = wid * EPW

    bufs = ((rows0, wba0, wbb0, dstca0, dstcb0, sg0, ss0),
            (rows1, wba1, wbb1, dstca1, dstcb1, sg1, ss1))

    pltpu.sync_copy(a_hbm, a_v)
    pltpu.sync_copy(b_hbm, b_v)

    def super_load(sidx):
        o = pl.multiple_of(base + sidx * SUP, 8)
        pltpu.sync_copy(src_hbm.at[pl.ds(o, SUP)], srcb)
        pltpu.sync_copy(dst_hbm.at[pl.ds(o, SUP)], dstb)

    super_load(0)

    zv = jnp.zeros((16,), jnp.float32)

    def _zrow(i, _):
        rows0[i // 8, pl.ds((i % 8) * 16, 16)] = zv
        return 0

    lax.fori_loop(0, CH * 8, _zrow, 0)

    def _zden(i, _):
        zden[pl.ds(i * 16, 16)] = zv
        return 0

    lax.fori_loop(0, 64, _zden, 0)

    r0 = pl.multiple_of(jnp.minimum(s_id * RPT, N - RPT), 8)
    for t in range(7):
        pltpu.sync_copy(rows0, acc_sh.at[pl.ds(r0 + t * CH, CH)])
    pltpu.sync_copy(rows0.at[pl.ds(0, 72)], acc_sh.at[pl.ds(r0 + 560, 72)])

    @pl.when(s_id < 10)
    def _():
        pltpu.sync_copy(zden.at[pl.ds(0, 1000)],
                        den_sh.at[pl.ds(s_id * 1000, 1000)])

    plsc.subcore_barrier()

    def wcompute(cj, wba, wbb, dstca, dstcb):
        q = (cj % CPS) * CH
        for g in range(CH // 16):
            s16 = srcb[pl.ds(q + g * 16, 16)]
            d16 = dstb[pl.ds(q + g * 16, 16)]
            if g < 3:
                dstca[pl.ds(g * 16, 16)] = d16
            else:
                dstcb[pl.ds((g - 3) * 16, 16)] = d16
            a_s = plsc.load_gather(a_v, [s16])
            a_d = plsc.load_gather(a_v, [d16])
            b_d = plsc.load_gather(b_v, [d16])
            w = jnp.exp(_leaky(a_s + b_d) - _leaky(a_d + b_d))
            if g < 3:
                wba[pl.ds(g * 16, 16)] = w
            else:
                wbb[pl.ds((g - 3) * 16, 16)] = w

    def gather_desc(cj, rows, sg):
        q = pl.multiple_of((cj % CPS) * CH, 8)
        return pltpu.make_async_copy(h_hbm.at[srcb.at[pl.ds(q, CH)]],
                                     rows, sg)

    def scale_half(rows, wb, glo, ghi):
        @plsc.parallel_loop(glo * 16, ghi * 16, 1, unroll=4)
        def _(k):
            wk = plsc.load_gather(wb, [jnp.full((16,), k - glo * 16,
                                                jnp.int32)])
            for qq in range(DIM // 16):
                sl = pl.ds(qq * 16, 16)
                rows[k, sl] = rows[k, sl] * wk

    def scatter_waits(rows, wba, wbb, dstca, dstcb, ss):
        pltpu.make_async_copy(rows.at[pl.ds(0, 48)],
                              acc_sh.at[dstca], ss).wait()
        pltpu.make_async_copy(rows.at[pl.ds(48, 32)],
                              acc_sh.at[dstcb], ss).wait()
        pltpu.make_async_copy(wba, den_sh.at[dstca], ss).wait()
        pltpu.make_async_copy(wbb, den_sh.at[dstcb], ss).wait()

    def iter_body(cj, P, first=False, has_next=True, may_bound=True):
        rowsP, wbaP, wbbP, dstcaP, dstcbP, sgP, ssP = bufs[P]
        rowsQ, wbaQ, wbbQ, dstcaQ, dstcbQ, sgQ, ssQ = bufs[1 - P]
        if not first:
            scatter_waits(rowsQ, wbaQ, wbbQ, dstcaQ, dstcbQ, ssQ)
        if may_bound:
            bound = ((cj + 1) % CPS) == 0

            @pl.when(bound)
            def _():
                gather_desc(cj, rowsP, sgP).wait()
                super_load((cj + 1) // CPS)

            if has_next:
                pltpu.async_copy(h_hbm.at[srcb.at[
                    pl.ds(pl.multiple_of(((cj + 1) % CPS) * CH, 8), CH)]],
                    rowsQ, sgQ)

            @pl.when(jnp.logical_not(bound))
            def _():
                gather_desc(cj, rowsP, sgP).wait()
        else:
            if has_next:
                pltpu.async_copy(h_hbm.at[srcb.at[
                    pl.ds(pl.multiple_of(((cj + 1) % CPS) * CH, 8), CH)]],
                    rowsQ, sgQ)
            gather_desc(cj, rowsP, sgP).wait()
        scale_half(rowsP, wbaP, 0, 3)
        pltpu.async_copy(rowsP.at[pl.ds(0, 48)], acc_sh.at[dstcaP],
                         ssP, add=True)
        scale_half(rowsP, wbbP, 3, 5)
        pltpu.async_copy(rowsP.at[pl.ds(48, 32)], acc_sh.at[dstcbP],
                         ssP, add=True)
        pltpu.async_copy(wbaP, den_sh.at[dstcaP], ssP, add=True)
        pltpu.async_copy(wbbP, den_sh.at[dstcbP], ssP, add=True)
        if has_next:
            wcompute(cj + 1, wbaQ, wbbQ, dstcaQ, dstcbQ)

    wcompute(0, wba0, wbb0, dstca0, dstcb0)
    pltpu.async_copy(h_hbm.at[srcb.at[pl.ds(0, CH)]], rows0, sg0)
    iter_body(0, 0, first=True, may_bound=False)

    def _pair(p, _):
        iter_body(2 * p + 1, 1)
        iter_body(2 * p + 2, 0)
        return 0

    lax.fori_loop(0, (NCH - 3) // 2, _pair, 0)

    iter_body(NCH - 2, 1, may_bound=False)
    iter_body(NCH - 1, 0, has_next=False, may_bound=False)
    scatter_waits(rows0, wba0, wbb0, dstca0, dstcb0, ss0)

    plsc.subcore_barrier()

    for t in range(7):
        pltpu.sync_copy(acc_sh.at[pl.ds(r0 + t * CH, CH)], rows0)
        pltpu.sync_copy(rows0, acc_out.at[c_id, pl.ds(r0 + t * CH, CH)])
    pltpu.sync_copy(acc_sh.at[pl.ds(r0 + 560, 72)], rows0.at[pl.ds(0, 72)])
    pltpu.sync_copy(rows0.at[pl.ds(0, 72)], acc_out.at[c_id, pl.ds(r0 + 560, 72)])

    @pl.when(s_id < 10)
    def _():
        pltpu.sync_copy(den_sh.at[pl.ds(s_id * 1000, 1000)],
                        zden.at[pl.ds(0, 1000)])
        pltpu.sync_copy(zden.at[pl.ds(0, 1000)],
                        den_out.at[pl.ds(c_id * N + s_id * 1000, 1000)])


_sc_edge_pass = functools.partial(
    pl.kernel,
    out_type=(jax.ShapeDtypeStruct((NUM_CORES, N, DIM), jnp.float32),
              jax.ShapeDtypeStruct((NUM_CORES * N,), jnp.float32)),
    mesh=plsc.VectorSubcoreMesh(core_axis_name="c", subcore_axis_name="s"),
    scratch_types=[
        pltpu.VMEM((N,), jnp.float32),
        pltpu.VMEM((N,), jnp.float32),
        pltpu.VMEM((SUP,), jnp.int32),
        pltpu.VMEM((SUP,), jnp.int32),
        pltpu.VMEM((48,), jnp.int32),
        pltpu.VMEM((32,), jnp.int32),
        pltpu.VMEM((48,), jnp.int32),
        pltpu.VMEM((32,), jnp.int32),
        pltpu.VMEM((48,), jnp.float32),
        pltpu.VMEM((32,), jnp.float32),
        pltpu.VMEM((48,), jnp.float32),
        pltpu.VMEM((32,), jnp.float32),
        pltpu.VMEM((CH, DIM), jnp.float32),
        pltpu.VMEM((CH, DIM), jnp.float32),
        pltpu.VMEM((1024,), jnp.float32),
        pltpu.VMEM_SHARED((N, DIM), jnp.float32),
        pltpu.VMEM_SHARED((N,), jnp.float32),
        pltpu.SemaphoreType.DMA,
        pltpu.SemaphoreType.DMA,
        pltpu.SemaphoreType.DMA,
        pltpu.SemaphoreType.DMA,
    ],
    compiler_params=pltpu.CompilerParams(needs_layout_passes=False),
)(_sc_body)


def kernel(z, edge_index, W, att_src, att_dst, bias):
    h, a, b, src, dst = pl.pallas_call(
        _prep_body,
        out_shape=(jax.ShapeDtypeStruct((N, DIM), jnp.float32),
                   jax.ShapeDtypeStruct((N, 1), jnp.float32),
                   jax.ShapeDtypeStruct((N, 1), jnp.float32),
                   jax.ShapeDtypeStruct((E,), jnp.int32),
                   jax.ShapeDtypeStruct((E,), jnp.int32)),
    )(z, W, att_src.reshape(1, DIM), att_dst.reshape(1, DIM), edge_index)

    acc, den = _sc_edge_pass(h, a.reshape(N), b.reshape(N), src, dst)

    out = pl.pallas_call(
        _combine_body,
        out_shape=jax.ShapeDtypeStruct((N, DIM), jnp.float32),
    )(h, acc, den.reshape(NUM_CORES, N, 1), bias.reshape(1, DIM))
    return out
